# baseline (speedup 1.0000x reference)
import functools
import jax
import jax.numpy as jnp
from jax import lax
from jax.experimental import pallas as pl
from jax.experimental.pallas import tpu as pltpu
from jax.experimental.pallas import tpu_sc as plsc

_B, _L, _LOC_MAX, _EMB = 4, 50, 2000, 16
_SU, _SL, _TU, _TL = 100.0, 0.0, 500.0, 0.0
_NPAIR = _B * _L
_NW = 32
_KMAX = 7

_DNUMS = lax.GatherDimensionNumbers(
    offset_dims=(), collapsed_slice_dims=(0,), start_index_map=(0,))


def _splat(x, m):
    idxs = jnp.full((16, 1), m, jnp.int32)
    return lax.gather(x, idxs, _DNUMS, (1,),
                      mode=lax.GatherScatterMode.PROMISE_IN_BOUNDS)


def _sc_body(idxw_hbm, vfw_hbm, vecw_hbm, esl_hbm, esu_hbm, etl_hbm, etu_hbm,
             mat2_hbm, out_hbm,
             esl_v, esu_v, etl_v, etu_v,
             idx16_v, vf16_v, vec16_v, rows_v, out_stage, gsem, osems):
    wid = lax.axis_index("s") * 2 + lax.axis_index("c")

    pltpu.sync_copy(idxw_hbm.at[wid], idx16_v)
    pltpu.sync_copy(vfw_hbm.at[wid], vf16_v)
    pltpu.sync_copy(vecw_hbm.at[wid], vec16_v)
    pltpu.sync_copy(esl_hbm, esl_v)
    pltpu.sync_copy(esu_hbm, esu_v)
    pltpu.sync_copy(etl_hbm, etl_v)
    pltpu.sync_copy(etu_hbm, etu_v)

    pltpu.async_copy(mat2_hbm.at[idx16_v], rows_v, gsem).wait()

    esl0 = esl_v[0, :]
    esl1 = esl_v[1, :]
    esu0 = esu_v[0, :]
    esu1 = esu_v[1, :]
    etl0 = etl_v[0, :]
    etl1 = etl_v[1, :]
    etu0 = etu_v[0, :]
    etu1 = etu_v[1, :]
    vfw = vf16_v[...]
    vecw = vec16_v[...]

    for k in range(_KMAX):
        p = wid + 32 * k

        @pl.when(p < _NPAIR)
        def _():
            v = _splat(vfw, k)
            t = _splat(vecw, k)

            esl = esl0 + v * (esl1 - esl0)
            esu = esu0 + v * (esu1 - esu0)
            etl = etl0 + v * (etl1 - etl0)
            etu = etu0 + v * (etu1 - etu0)
            base = esl + etl + (etu - etl) * (t * (1.0 / _TU))
            coef = (esu - esl) * (v * (1.0 / _SU))

            buf = k % 2

            def jblock(j0, _):
                rv = rows_v[k, pl.ds(j0 * 16, 16)]
                for m in range(16):
                    out_stage[buf, j0 * 16 + m] = base + coef * _splat(rv, m)
                return 0

            lax.fori_loop(0, _LOC_MAX // 16, jblock, 0)
            pltpu.sync_copy(out_stage.at[buf], out_hbm.at[p])


def kernel(traj_loc, mat2, vec, traj_len, emb_su, emb_sl, emb_tu, emb_tl):
    idx = (traj_loc.reshape(-1) - 1).astype(jnp.int32)
    vf = (jnp.arange(_L)[None, :] < traj_len[:, None]).astype(
        jnp.float32).reshape(-1)
    vecv = vec.reshape(-1).astype(jnp.float32)

    m8 = jnp.minimum(
        jnp.arange(_NW)[:, None] + 32 * jnp.arange(8)[None, :],
        _NPAIR - 1)
    m16 = jnp.minimum(
        jnp.arange(_NW)[:, None] + 32 * jnp.arange(16)[None, :],
        _NPAIR - 1)
    idxw = idx[m8]
    vfw = vf[m16]
    vecw = vecv[m16]

    mesh = plsc.VectorSubcoreMesh(core_axis_name="c", subcore_axis_name="s")
    run = functools.partial(
        pl.kernel,
        out_type=jax.ShapeDtypeStruct((_NPAIR, _LOC_MAX, _EMB), jnp.float32),
        mesh=mesh,
        compiler_params=pltpu.CompilerParams(use_tc_tiling_on_sc=False),
        scratch_types=[
            pltpu.VMEM((2, _EMB), jnp.float32),
            pltpu.VMEM((2, _EMB), jnp.float32),
            pltpu.VMEM((2, _EMB), jnp.float32),
            pltpu.VMEM((2, _EMB), jnp.float32),
            pltpu.VMEM((8,), jnp.int32),
            pltpu.VMEM((16,), jnp.float32),
            pltpu.VMEM((16,), jnp.float32),
            pltpu.VMEM((8, _LOC_MAX), jnp.float32),
            pltpu.VMEM((2, _LOC_MAX, _EMB), jnp.float32),
            pltpu.SemaphoreType.DMA,
            pltpu.SemaphoreType.DMA((_KMAX,)),
        ],
    )(_sc_body)
    out = run(idxw, vfw, vecw, emb_sl, emb_su, emb_tl, emb_tu, mat2)
    return out.reshape(_B, _L, _LOC_MAX, _EMB)


# speedup vs baseline: 1.0086x; 1.0086x over previous
import functools
import jax
import jax.numpy as jnp
from jax import lax
from jax.experimental import pallas as pl
from jax.experimental.pallas import tpu as pltpu
from jax.experimental.pallas import tpu_sc as plsc

_B, _L, _LOC_MAX, _EMB = 4, 50, 2000, 16
_SU, _SL, _TU, _TL = 100.0, 0.0, 500.0, 0.0
_NPAIR = _B * _L
_NW = 32
_KMAX = 7

_DNUMS = lax.GatherDimensionNumbers(
    offset_dims=(), collapsed_slice_dims=(0,), start_index_map=(0,))


def _splat(x, m):
    idxs = jnp.full((16, 1), m, jnp.int32)
    return lax.gather(x, idxs, _DNUMS, (1,),
                      mode=lax.GatherScatterMode.PROMISE_IN_BOUNDS)


def _sc_body(idxw_hbm, vfw_hbm, vecw_hbm, esl_hbm, esu_hbm, etl_hbm, etu_hbm,
             mat2_hbm, out_hbm,
             esl_v, esu_v, etl_v, etu_v,
             idx16_v, vf16_v, vec16_v, rows_v, out_stage, gsem, osems):
    wid = lax.axis_index("s") * 2 + lax.axis_index("c")
    start = jnp.where(wid < 8, wid * 7, 56 + 6 * (wid - 8))
    nw = jnp.where(wid < 8, 7, 6)

    pltpu.sync_copy(idxw_hbm.at[wid], idx16_v)
    pltpu.sync_copy(vfw_hbm.at[wid], vf16_v)
    pltpu.sync_copy(vecw_hbm.at[wid], vec16_v)
    pltpu.sync_copy(esl_hbm, esl_v)
    pltpu.sync_copy(esu_hbm, esu_v)
    pltpu.sync_copy(etl_hbm, etl_v)
    pltpu.sync_copy(etu_hbm, etu_v)

    pltpu.async_copy(mat2_hbm.at[idx16_v], rows_v, gsem).wait()

    esl0 = esl_v[0, :]
    esl1 = esl_v[1, :]
    esu0 = esu_v[0, :]
    esu1 = esu_v[1, :]
    etl0 = etl_v[0, :]
    etl1 = etl_v[1, :]
    etu0 = etu_v[0, :]
    etu1 = etu_v[1, :]
    vfw = vf16_v[...]
    vecw = vec16_v[...]

    for k0, gsize in ((0, 3), (3, 3), (6, 1)):
        for k in range(k0, k0 + gsize):
            buf = k - k0

            @pl.when(k < nw)
            def _():
                v = _splat(vfw, k)
                t = _splat(vecw, k)

                esl = esl0 + v * (esl1 - esl0)
                esu = esu0 + v * (esu1 - esu0)
                etl = etl0 + v * (etl1 - etl0)
                etu = etu0 + v * (etu1 - etu0)
                base = esl + etl + (etu - etl) * (t * (1.0 / _TU))
                coef = (esu - esl) * (v * (1.0 / _SU))

                def jblock(j0, _):
                    rv = rows_v[k, pl.ds(j0 * 16, 16)]
                    for m in range(16):
                        out_stage[buf, j0 * 16 + m] = (
                            base + coef * _splat(rv, m))
                    return 0

                lax.fori_loop(0, _LOC_MAX // 16, jblock, 0)

        @pl.when(k0 < nw)
        def _():
            pltpu.sync_copy(out_stage.at[pl.ds(0, gsize)],
                            out_hbm.at[pl.ds(start + k0, gsize)])


def kernel(traj_loc, mat2, vec, traj_len, emb_su, emb_sl, emb_tu, emb_tl):
    idx = (traj_loc.reshape(-1) - 1).astype(jnp.int32)
    vf = (jnp.arange(_L)[None, :] < traj_len[:, None]).astype(
        jnp.float32).reshape(-1)
    vecv = vec.reshape(-1).astype(jnp.float32)

    starts = jnp.where(jnp.arange(_NW) < 8,
                       jnp.arange(_NW) * 7,
                       56 + 6 * (jnp.arange(_NW) - 8))
    m8 = jnp.minimum(starts[:, None] + jnp.arange(8)[None, :],
                     _NPAIR - 1)
    m16 = jnp.minimum(starts[:, None] + jnp.arange(16)[None, :],
                      _NPAIR - 1)
    idxw = idx[m8]
    vfw = vf[m16]
    vecw = vecv[m16]

    mesh = plsc.VectorSubcoreMesh(core_axis_name="c", subcore_axis_name="s")
    run = functools.partial(
        pl.kernel,
        out_type=jax.ShapeDtypeStruct((_NPAIR, _LOC_MAX, _EMB), jnp.float32),
        mesh=mesh,
        compiler_params=pltpu.CompilerParams(use_tc_tiling_on_sc=False),
        scratch_types=[
            pltpu.VMEM((2, _EMB), jnp.float32),
            pltpu.VMEM((2, _EMB), jnp.float32),
            pltpu.VMEM((2, _EMB), jnp.float32),
            pltpu.VMEM((2, _EMB), jnp.float32),
            pltpu.VMEM((8,), jnp.int32),
            pltpu.VMEM((16,), jnp.float32),
            pltpu.VMEM((16,), jnp.float32),
            pltpu.VMEM((8, _LOC_MAX), jnp.float32),
            pltpu.VMEM((3, _LOC_MAX, _EMB), jnp.float32),
            pltpu.SemaphoreType.DMA,
            pltpu.SemaphoreType.DMA((_KMAX,)),
        ],
    )(_sc_body)
    out = run(idxw, vfw, vecw, emb_sl, emb_su, emb_tl, emb_tu, mat2)
    return out.reshape(_B, _L, _LOC_MAX, _EMB)


# speedup vs baseline: 1.0102x; 1.0017x over previous
import functools
import jax
import jax.numpy as jnp
from jax import lax
from jax.experimental import pallas as pl
from jax.experimental.pallas import tpu as pltpu
from jax.experimental.pallas import tpu_sc as plsc

_B, _L, _LOC_MAX, _EMB = 4, 50, 2000, 16
_SU, _SL, _TU, _TL = 100.0, 0.0, 500.0, 0.0
_NPAIR = _B * _L
_NW = 32
_KMAX = 7

_DNUMS = lax.GatherDimensionNumbers(
    offset_dims=(), collapsed_slice_dims=(0,), start_index_map=(0,))


def _splat(x, m):
    idxs = jnp.full((16, 1), m, jnp.int32)
    return lax.gather(x, idxs, _DNUMS, (1,),
                      mode=lax.GatherScatterMode.PROMISE_IN_BOUNDS)


def _sc_body(idxw_hbm, vfw_hbm, vecw_hbm, esl_hbm, esu_hbm, etl_hbm, etu_hbm,
             mat2_hbm, out_hbm,
             esl_v, esu_v, etl_v, etu_v,
             idx16_v, vf16_v, vec16_v, rows_v, out_stage, gsem):
    wid = lax.axis_index("s") * 2 + lax.axis_index("c")
    start = jnp.where(wid < 8, wid * 7, 56 + 6 * (wid - 8))
    nw = jnp.where(wid < 8, 7, 6)

    pltpu.sync_copy(idxw_hbm.at[wid], idx16_v)
    pltpu.sync_copy(vfw_hbm.at[wid], vf16_v)
    pltpu.sync_copy(vecw_hbm.at[wid], vec16_v)
    pltpu.sync_copy(esl_hbm, esl_v)
    pltpu.sync_copy(esu_hbm, esu_v)
    pltpu.sync_copy(etl_hbm, etl_v)
    pltpu.sync_copy(etu_hbm, etu_v)

    pltpu.async_copy(mat2_hbm.at[idx16_v], rows_v, gsem).wait()

    esl0 = esl_v[0, :]
    esl1 = esl_v[1, :]
    esu0 = esu_v[0, :]
    esu1 = esu_v[1, :]
    etl0 = etl_v[0, :]
    etl1 = etl_v[1, :]
    etu0 = etu_v[0, :]
    etu1 = etu_v[1, :]
    vfw = vf16_v[...]
    vecw = vec16_v[...]

    for k0, gsize in ((0, 3), (3, 3), (6, 1)):
        for k in range(k0, k0 + gsize):
            buf = k - k0

            @pl.when(k < nw)
            def _():
                v = _splat(vfw, k)
                t = _splat(vecw, k)

                esl = esl0 + v * (esl1 - esl0)
                esu = esu0 + v * (esu1 - esu0)
                etl = etl0 + v * (etl1 - etl0)
                etu = etu0 + v * (etu1 - etu0)
                base = esl + etl + (etu - etl) * (t * (1.0 / _TU))
                coef = (esu - esl) * (v * (1.0 / _SU))

                def jblock(j0, _):
                    rv = rows_v[k, pl.ds(j0 * 16, 16)]
                    for m in range(16):
                        out_stage[buf, j0 * 16 + m] = (
                            base + coef * _splat(rv, m))
                    return 0

                lax.fori_loop(0, _LOC_MAX // 16, jblock, 0)

        @pl.when(k0 < nw)
        def _():
            pltpu.sync_copy(out_stage.at[pl.ds(0, gsize)],
                            out_hbm.at[pl.ds(start + k0, gsize)])


def kernel(traj_loc, mat2, vec, traj_len, emb_su, emb_sl, emb_tu, emb_tl):
    idx = (traj_loc.reshape(-1) - 1).astype(jnp.int32)
    vf = (jnp.arange(_L)[None, :] < traj_len[:, None]).astype(
        jnp.float32).reshape(-1)
    vecv = vec.reshape(-1).astype(jnp.float32)

    starts = jnp.where(jnp.arange(_NW) < 8,
                       jnp.arange(_NW) * 7,
                       56 + 6 * (jnp.arange(_NW) - 8))
    m8 = jnp.minimum(starts[:, None] + jnp.arange(8)[None, :],
                     _NPAIR - 1)
    m16 = jnp.minimum(starts[:, None] + jnp.arange(16)[None, :],
                      _NPAIR - 1)
    idxw = idx[m8]
    vfw = vf[m16]
    vecw = vecv[m16]

    mesh = plsc.VectorSubcoreMesh(core_axis_name="c", subcore_axis_name="s")
    run = functools.partial(
        pl.kernel,
        out_type=jax.ShapeDtypeStruct((_NPAIR, _LOC_MAX, _EMB), jnp.float32),
        mesh=mesh,
        compiler_params=pltpu.CompilerParams(use_tc_tiling_on_sc=False),
        scratch_types=[
            pltpu.VMEM((2, _EMB), jnp.float32),
            pltpu.VMEM((2, _EMB), jnp.float32),
            pltpu.VMEM((2, _EMB), jnp.float32),
            pltpu.VMEM((2, _EMB), jnp.float32),
            pltpu.VMEM((8,), jnp.int32),
            pltpu.VMEM((16,), jnp.float32),
            pltpu.VMEM((16,), jnp.float32),
            pltpu.VMEM((8, _LOC_MAX), jnp.float32),
            pltpu.VMEM((3, _LOC_MAX, _EMB), jnp.float32),
            pltpu.SemaphoreType.DMA,
        ],
    )(_sc_body)
    out = run(idxw, vfw, vecw, emb_sl, emb_su, emb_tl, emb_tu, mat2)
    return out.reshape(_B, _L, _LOC_MAX, _EMB)
